# baseline (speedup 1.0000x reference)
import numpy as np
import jax
import jax.numpy as jnp
from jax.experimental import pallas as pl
from jax.experimental.pallas import tpu as pltpu

_C_IN = 3
_C1 = 4
_C2 = 4
_K = 5
_H = 32
_H1 = 28
_P1 = 14
_H2 = 10
_P2 = 5
_F = _C2 * _P2 * _P2
_N_B = 1024
_G = 4

_NG = _H1 // _G
_M1 = 2 * _G * _P1
_KS1 = (_G + _K - 1) * _H
_M2 = 2 * _H2 * _P2
_KS2 = _P1 * _P1


def _factors():
    f32 = np.float32
    dy = np.arange(_K)
    a1 = (np.arange(_G + _K - 1)[None, :, None]
          == np.arange(_G)[None, None, :] + dy[:, None, None]).astype(f32)
    b1 = (np.arange(_H)[None, :, None, None]
          == 2 * np.arange(_P1)[None, None, None, :]
          + np.arange(2)[None, None, :, None]
          + dy[:, None, None, None]).astype(f32)
    f1 = np.einsum("djr,eupx->deprxju", a1, b1).reshape(_K * _K, _M1, _KS1)
    a2 = (np.arange(_P1)[None, :, None]
          == np.arange(_H2)[None, None, :] + dy[:, None, None]).astype(f32)
    b2 = (np.arange(_P1)[None, :, None, None]
          == 2 * np.arange(_P2)[None, None, None, :]
          + np.arange(2)[None, None, :, None]
          + dy[:, None, None, None]).astype(f32)
    f2 = np.einsum("dyr,eupx->deprxyu", a2, b2).reshape(_K * _K, _M2, _KS2)
    return f1, f2


_F1, _F2 = _factors()


def _encoder_body(x_ref, f1_ref, f2_ref, w1_ref, b1_ref, w2_ref, b2_ref,
                  wfc_ref, bfc_ref, muw_ref, bmu_ref, vaw_ref, bva_ref,
                  mu_ref, lv_ref, p1_ref, w1s_ref, w2s_ref):
    f32 = jnp.float32

    @pl.when(pl.program_id(0) == 0)
    def _build():
        for o in range(_C1):
            for c in range(_C_IN):
                base = (o * _C_IN + c) * (_K * _K)
                acc = w1_ref[base] * f1_ref[0]
                for k in range(1, _K * _K):
                    acc = acc + w1_ref[base + k] * f1_ref[k]
                w1s_ref[o, c] = acc
        for o in range(_C2):
            for c in range(_C1):
                base = (o * _C1 + c) * (_K * _K)
                acc = w2_ref[base] * f2_ref[0]
                for k in range(1, _K * _K):
                    acc = acc + w2_ref[base + k] * f2_ref[k]
                w2s_ref[o, c] = acc

    for g in range(_NG):
        h = None
        for c in range(_C_IN):
            base = c * (_H * _H) + g * _G * _H
            d = jnp.dot(w1s_ref[:, c].reshape(_C1 * _M1, _KS1),
                        x_ref[base:base + _KS1, :],
                        preferred_element_type=f32)
            h = d if h is None else h + d
        for o in range(_C1):
            ho = h[o * _M1:(o + 1) * _M1, :]
            vo = jnp.maximum(
                jnp.maximum(ho[:_M1 // 2, :], ho[_M1 // 2:, :]) + b1_ref[o],
                0.0)
            r0 = o * (_P1 * _P1) + 2 * g * _P1
            p1_ref[r0:r0 + _P1, :] = (
                jnp.maximum(vo[0 * _P1:1 * _P1], vo[1 * _P1:2 * _P1]))
            p1_ref[r0 + _P1:r0 + 2 * _P1, :] = (
                jnp.maximum(vo[2 * _P1:3 * _P1], vo[3 * _P1:4 * _P1]))

    fs = []
    for o in range(_C2):
        h2 = None
        for c in range(_C1):
            d = jnp.dot(w2s_ref[o, c], p1_ref[c * _KS2:(c + 1) * _KS2, :],
                        preferred_element_type=f32)
            h2 = d if h2 is None else h2 + d
        vo = jnp.maximum(
            jnp.maximum(h2[:_M2 // 2, :], h2[_M2 // 2:, :]) + b2_ref[o],
            0.0)
        for k in range(_P2):
            fs.append(jnp.maximum(vo[(2 * k) * _P2:(2 * k + 1) * _P2],
                                  vo[(2 * k + 1) * _P2:(2 * k + 2) * _P2]))
    f = jnp.concatenate(fs, axis=0)

    hid = jnp.dot(wfc_ref[...], f, preferred_element_type=f32) + bfc_ref[...]
    hid = jnp.maximum(hid, 0.0)
    mu_ref[...] = jnp.dot(muw_ref[...], hid,
                          preferred_element_type=f32) + bmu_ref[...]
    lv_ref[...] = jnp.dot(vaw_ref[...], hid,
                          preferred_element_type=f32) + bva_ref[...]


def kernel(state, w1, b1, w2, b2, fcw, fcb, muw, mub, vaw, vab):
    f32 = jnp.float32
    in_shape = state.shape
    xt = state.astype(f32).reshape(-1, _C_IN * _H * _H).T
    B = xt.shape[1]
    L = muw.shape[0]

    nt = pl.cdiv(B, _N_B)
    bp = nt * _N_B
    if bp != B:
        xt = jnp.pad(xt, ((0, 0), (0, bp - B)))

    smem = pl.BlockSpec(memory_space=pltpu.MemorySpace.SMEM)
    full2 = lambda t: (0, 0)
    out_shapes = (jax.ShapeDtypeStruct((L, bp), f32),
                  jax.ShapeDtypeStruct((L, bp), f32))
    mu, lv = pl.pallas_call(
        _encoder_body,
        grid=(nt,),
        in_specs=[
            pl.BlockSpec((_C_IN * _H * _H, _N_B), lambda t: (0, t)),
            pl.BlockSpec((_K * _K, _M1, _KS1), lambda t: (0, 0, 0)),
            pl.BlockSpec((_K * _K, _M2, _KS2), lambda t: (0, 0, 0)),
            smem, smem, smem, smem,
            pl.BlockSpec((_F, _F), full2),
            pl.BlockSpec((_F, 1), full2),
            pl.BlockSpec((L, _F), full2),
            pl.BlockSpec((L, 1), full2),
            pl.BlockSpec((L, _F), full2),
            pl.BlockSpec((L, 1), full2),
        ],
        out_specs=[pl.BlockSpec((L, _N_B), lambda t: (0, t)),
                   pl.BlockSpec((L, _N_B), lambda t: (0, t))],
        out_shape=out_shapes,
        scratch_shapes=[
            pltpu.VMEM((_C1 * _KS2, _N_B), f32),
            pltpu.VMEM((_C1, _C_IN, _M1, _KS1), f32),
            pltpu.VMEM((_C2, _C1, _M2, _KS2), f32),
        ],
        compiler_params=pltpu.CompilerParams(
            dimension_semantics=("arbitrary",),
            vmem_limit_bytes=48 * 1024 * 1024),
    )(xt, jnp.asarray(_F1), jnp.asarray(_F2),
      w1.astype(f32).reshape(-1), b1.astype(f32),
      w2.astype(f32).reshape(-1), b2.astype(f32),
      fcw.astype(f32), fcb.astype(f32)[:, None],
      muw.astype(f32), mub.astype(f32)[:, None],
      vaw.astype(f32), vab.astype(f32)[:, None])

    mu = mu[:, :B].T.reshape(*in_shape[:-3], L)
    log_var = lv[:, :B].T.reshape(*in_shape[:-3], L)
    return mu, log_var


# speedup vs baseline: 1.2508x; 1.2508x over previous
import numpy as np
import jax
import jax.numpy as jnp
from jax.experimental import pallas as pl
from jax.experimental.pallas import tpu as pltpu

_C_IN = 3
_C1 = 4
_C2 = 4
_K = 5
_H = 32
_H1 = 28
_P1 = 14
_H2 = 10
_P2 = 5
_F = _C2 * _P2 * _P2
_N_B = 1024
_G = 4

_NG = _H1 // _G
_M1 = 2 * _G * _P1
_KS1 = (_G + _K - 1) * _H
_M2 = 2 * _H2 * _P2
_KS2 = _P1 * _P1


def _factors():
    f32 = np.float32
    dy = np.arange(_K)
    a1 = (np.arange(_G + _K - 1)[None, :, None]
          == np.arange(_G)[None, None, :] + dy[:, None, None]).astype(f32)
    b1 = (np.arange(_H)[None, :, None, None]
          == 2 * np.arange(_P1)[None, None, None, :]
          + np.arange(2)[None, None, :, None]
          + dy[:, None, None, None]).astype(f32)
    f1 = np.einsum("djr,eupx->deprxju", a1, b1).reshape(_K * _K, _M1, _KS1)
    a2 = (np.arange(_P1)[None, :, None]
          == np.arange(_H2)[None, None, :] + dy[:, None, None]).astype(f32)
    b2 = (np.arange(_P1)[None, :, None, None]
          == 2 * np.arange(_P2)[None, None, None, :]
          + np.arange(2)[None, None, :, None]
          + dy[:, None, None, None]).astype(f32)
    f2 = np.einsum("dyr,eupx->deprxyu", a2, b2).reshape(_K * _K, _M2, _KS2)
    return f1, f2


_F1, _F2 = _factors()


def _encoder_body(x_ref, w1_ref, b1_ref, w2_ref, b2_ref,
                  wfc_ref, bfc_ref, muw_ref, bmu_ref, vaw_ref, bva_ref,
                  mu_ref, lv_ref, p1_ref):
    f32 = jnp.float32

    for g in range(_NG):
        h = None
        for c in range(_C_IN):
            base = c * (_H * _H) + g * _G * _H
            d = jnp.dot(w1_ref[:, c].reshape(_C1 * _M1, _KS1),
                        x_ref[base:base + _KS1, :],
                        preferred_element_type=f32)
            h = d if h is None else h + d
        for o in range(_C1):
            ho = h[o * _M1:(o + 1) * _M1, :]
            vo = jnp.maximum(
                jnp.maximum(ho[:_M1 // 2, :], ho[_M1 // 2:, :]) + b1_ref[o],
                0.0)
            r0 = o * (_P1 * _P1) + 2 * g * _P1
            p1_ref[r0:r0 + _P1, :] = (
                jnp.maximum(vo[0 * _P1:1 * _P1], vo[1 * _P1:2 * _P1]))
            p1_ref[r0 + _P1:r0 + 2 * _P1, :] = (
                jnp.maximum(vo[2 * _P1:3 * _P1], vo[3 * _P1:4 * _P1]))

    fs = []
    for o in range(_C2):
        h2 = None
        for c in range(_C1):
            d = jnp.dot(w2_ref[o, c], p1_ref[c * _KS2:(c + 1) * _KS2, :],
                        preferred_element_type=f32)
            h2 = d if h2 is None else h2 + d
        vo = jnp.maximum(
            jnp.maximum(h2[:_M2 // 2, :], h2[_M2 // 2:, :]) + b2_ref[o],
            0.0)
        for k in range(_P2):
            fs.append(jnp.maximum(vo[(2 * k) * _P2:(2 * k + 1) * _P2],
                                  vo[(2 * k + 1) * _P2:(2 * k + 2) * _P2]))
    f = jnp.concatenate(fs, axis=0)

    hid = jnp.dot(wfc_ref[...], f, preferred_element_type=f32) + bfc_ref[...]
    hid = jnp.maximum(hid, 0.0)
    mu_ref[...] = jnp.dot(muw_ref[...], hid,
                          preferred_element_type=f32) + bmu_ref[...]
    lv_ref[...] = jnp.dot(vaw_ref[...], hid,
                          preferred_element_type=f32) + bva_ref[...]


def kernel(state, w1, b1, w2, b2, fcw, fcb, muw, mub, vaw, vab):
    f32 = jnp.float32
    in_shape = state.shape
    xt = state.astype(f32).reshape(-1, _C_IN * _H * _H).T
    B = xt.shape[1]
    L = muw.shape[0]

    nt = pl.cdiv(B, _N_B)
    bp = nt * _N_B
    if bp != B:
        xt = jnp.pad(xt, ((0, 0), (0, bp - B)))

    w1g = (w1.astype(f32).reshape(_C1 * _C_IN, _K * _K)
           @ jnp.asarray(_F1.reshape(_K * _K, -1))
           ).reshape(_C1, _C_IN, _M1, _KS1)
    w2g = (w2.astype(f32).reshape(_C2 * _C2, _K * _K)
           @ jnp.asarray(_F2.reshape(_K * _K, -1))
           ).reshape(_C2, _C2, _M2, _KS2)

    bfcr = fcb.astype(f32)[:, None]

    smem = pl.BlockSpec(memory_space=pltpu.MemorySpace.SMEM)
    full2 = lambda t: (0, 0)
    out = pl.pallas_call(
        _encoder_body,
        grid=(nt,),
        in_specs=[
            pl.BlockSpec((_C_IN * _H * _H, _N_B), lambda t: (0, t)),
            pl.BlockSpec((_C1, _C_IN, _M1, _KS1), lambda t: (0, 0, 0, 0)),
            smem,
            pl.BlockSpec((_C2, _C1, _M2, _KS2), lambda t: (0, 0, 0, 0)),
            smem,
            pl.BlockSpec((_F, _F), full2),
            pl.BlockSpec((_F, 1), full2),
            pl.BlockSpec((L, _F), full2),
            pl.BlockSpec((L, 1), full2),
            pl.BlockSpec((L, _F), full2),
            pl.BlockSpec((L, 1), full2),
        ],
        out_specs=[pl.BlockSpec((L, _N_B), lambda t: (0, t)),
                   pl.BlockSpec((L, _N_B), lambda t: (0, t))],
        out_shape=(jax.ShapeDtypeStruct((L, bp), f32),
                   jax.ShapeDtypeStruct((L, bp), f32)),
        scratch_shapes=[pltpu.VMEM((_C1 * _KS2, _N_B), f32)],
        compiler_params=pltpu.CompilerParams(
            dimension_semantics=("parallel",),
            vmem_limit_bytes=40 * 1024 * 1024),
    )(xt, w1g, b1.astype(f32), w2g, b2.astype(f32),
      fcw.astype(f32), bfcr,
      muw.astype(f32), mub.astype(f32)[:, None],
      vaw.astype(f32), vab.astype(f32)[:, None])
    mu, lv = out

    mu = mu[:, :B].T.reshape(*in_shape[:-3], L)
    log_var = lv[:, :B].T.reshape(*in_shape[:-3], L)
    return mu, log_var
